# baseline (speedup 1.0000x reference)
import numpy as np
import jax
import jax.numpy as jnp
from jax import lax
from jax.experimental import pallas as pl
from jax.experimental.pallas import tpu as pltpu
from jax.experimental.pallas import tpu_sc as plsc

N = 10000
E = 320000
D = 128
NB = 8
NSYS = 8
RMAX = 5.0

NPTC = 10240
NROW = 4 * NPTC
TRASH = 4 * N
NW = 32
EPW = E // NW
CHUNK = 2000
NGRP = CHUNK // 16
RB = 4
ROWS_PER_TILE = NROW // 16


def _sc_edge_kernel(pos_h, types_h, src_h, dst_h, zeros_h, out_h,
                    pos, types, srcb, dstb, rbfb, rowb, racc):
    cid = lax.axis_index("c")
    sid = lax.axis_index("s")
    wid = sid * 2 + cid

    pltpu.sync_copy(zeros_h, racc.at[pl.ds(sid * ROWS_PER_TILE, ROWS_PER_TILE)])
    pltpu.sync_copy(pos_h, pos)
    pltpu.sync_copy(types_h, types)
    plsc.subcore_barrier()

    zero16 = jnp.zeros((16,), jnp.int32)
    one16 = jnp.full((16,), 1, jnp.int32)
    two16 = jnp.full((16,), 2, jnp.int32)
    pi_over_r = jnp.float32(np.pi / RMAX)
    two_over_pi = jnp.float32(2.0 / np.pi)
    half_pi = jnp.float32(np.pi / 2)
    rmaxf = jnp.float32(RMAX)
    trash16 = jnp.full((16,), TRASH, jnp.int32)

    base = wid * EPW

    def chunk_body(ci, c_in):
        cbase = pl.multiple_of(base + ci * CHUNK, 8)
        pltpu.sync_copy(src_h.at[pl.ds(cbase, CHUNK)], srcb)
        pltpu.sync_copy(dst_h.at[pl.ds(cbase, CHUNK)], dstb)

        def group(g8, c):
            g0 = g8 * 16
            sv = srcb[pl.ds(g0, 16)]
            dv = dstb[pl.ds(g0, 16)]
            xs = plsc.load_gather(pos, [sv, zero16])
            ys = plsc.load_gather(pos, [sv, one16])
            zs = plsc.load_gather(pos, [sv, two16])
            xd = plsc.load_gather(pos, [dv, zero16])
            yd = plsc.load_gather(pos, [dv, one16])
            zd = plsc.load_gather(pos, [dv, two16])
            tv = plsc.load_gather(types, [sv])

            dx = xd - xs
            dy = yd - ys
            dz = zd - zs
            d2 = dx * dx + dy * dy + dz * dz + jnp.float32(1e-12)
            yi = jnp.int32(0x5F3759DF) - lax.shift_right_arithmetic(
                plsc.bitcast(d2, jnp.int32), 1)
            ry = plsc.bitcast(yi, jnp.float32)
            ry = ry * (1.5 - 0.5 * d2 * ry * ry)
            ry = ry * (1.5 - 0.5 * d2 * ry * ry)
            ry = ry * (1.5 - 0.5 * d2 * ry * ry)
            d = d2 * ry
            keep = d < rmaxf
            dc = jnp.minimum(d, rmaxf)
            theta = dc * pi_over_r
            q = (theta * two_over_pi + 0.5).astype(jnp.int32)
            r = theta - q.astype(jnp.float32) * half_pi
            r2 = r * r
            sr = r + r * r2 * (jnp.float32(-1.6666654611e-1) + r2 *
                               (jnp.float32(8.3321608736e-3) + r2 *
                                jnp.float32(-1.9515295891e-4)))
            cr = 1.0 - 0.5 * r2 + r2 * r2 * (
                jnp.float32(4.166664568298827e-2) + r2 *
                (jnp.float32(-1.388731625493765e-3) + r2 *
                 jnp.float32(2.443315711809948e-5)))
            q1 = q == 1
            q0 = q == 0
            s1 = jnp.where(q0, sr, jnp.where(q1, cr, -sr))
            c1 = jnp.where(q0, cr, jnp.where(q1, -sr, -cr))
            g = (0.5 * (c1 + 1.0)) * ry
            twoc = c1 + c1

            mi = keep.astype(jnp.int32)
            pos_v = c + plsc.cumsum(mi) - 1
            pr = pos_v & jnp.full((16,), RB * 128 - 1, jnp.int32)
            sk_1 = s1
            sk = twoc * s1
            plsc.store_scatter(rbfb, [pr, zero16], s1 * g, mask=keep)
            plsc.store_scatter(rbfb, [pr, one16], sk * g, mask=keep)
            for k in range(2, NB):
                sk_1, sk = sk, twoc * sk - sk_1
                plsc.store_scatter(rbfb, [pr, jnp.full((16,), k, jnp.int32)],
                                   sk * g, mask=keep)

            rows_v = dv * 4 + tv
            plsc.store_scatter(rowb,
                               [lax.shift_right_logical(pr, 7),
                                pr & jnp.full((16,), 127, jnp.int32)],
                               rows_v, mask=keep)

            c2 = c + jnp.sum(mi)

            @pl.when(lax.shift_right_logical(c2, 7)
                     > lax.shift_right_logical(c, 7))
            def _():
                slot = lax.shift_right_logical(c, 7) & (RB - 1)
                pltpu.sync_copy(rbfb.at[pl.ds(slot * 128, 128)],
                                racc.at[rowb.at[slot]], add=True)

            return c2

        return lax.fori_loop(0, NGRP, group, c_in, unroll=4)

    cnt = lax.fori_loop(0, EPW // CHUNK, chunk_body, jnp.int32(0))

    @pl.when((cnt & 127) != 0)
    def _():
        lane = lax.iota(jnp.int32, 16)
        roundup = (cnt + 127) & jnp.int32(~127)
        for i in range(8):
            p = cnt + lane + i * 16
            prp = p & jnp.full((16,), RB * 128 - 1, jnp.int32)
            plsc.store_scatter(rowb,
                               [lax.shift_right_logical(prp, 7),
                                prp & jnp.full((16,), 127, jnp.int32)],
                               trash16, mask=p < roundup)
        slot = lax.shift_right_logical(cnt, 7) & (RB - 1)
        pltpu.sync_copy(rbfb.at[pl.ds(slot * 128, 128)],
                        racc.at[rowb.at[slot]], add=True)

    plsc.subcore_barrier()
    pltpu.sync_copy(racc.at[pl.ds(sid * ROWS_PER_TILE, ROWS_PER_TILE)],
                    out_h.at[cid, pl.ds(sid * ROWS_PER_TILE, ROWS_PER_TILE)])


def _sc_edge_pass(positions, types, src, dst):
    zeros = jnp.zeros((ROWS_PER_TILE, NB), jnp.float32)
    mesh = plsc.VectorSubcoreMesh(core_axis_name="c", subcore_axis_name="s")
    k = pl.kernel(
        _sc_edge_kernel,
        out_type=jax.ShapeDtypeStruct((2, NROW, NB), jnp.float32),
        mesh=mesh,
        compiler_params=pltpu.CompilerParams(needs_layout_passes=False,
                                             use_tc_tiling_on_sc=False),
        scratch_types=[
            pltpu.VMEM((N, 3), jnp.float32),
            pltpu.VMEM((N,), jnp.int32),
            pltpu.VMEM((CHUNK,), jnp.int32),
            pltpu.VMEM((CHUNK,), jnp.int32),
            pltpu.VMEM((RB * 128, NB), jnp.float32),
            pltpu.VMEM((RB, 128), jnp.int32),
            pltpu.VMEM_SHARED((NROW, NB), jnp.float32),
        ],
    )
    return k(positions, types, src, dst, zeros)


def _tc_tail_kernel(r2_ref, types_ref, batch_ref, te_ref, we_ref, wm_ref,
                    wu_ref, wo_ref, out_ref):
    i = pl.program_id(0)
    rblk = r2_ref[0] + r2_ref[1]
    hm4 = jnp.dot(te_ref[...], wm_ref[...], preferred_element_type=jnp.float32)
    we = we_ref[...]
    wflat = jnp.concatenate([we * hm4[t:t + 1, :] for t in range(4)], axis=0)
    agg = jnp.dot(rblk, wflat, preferred_element_type=jnp.float32)
    tv = types_ref[0]
    cols4 = lax.broadcasted_iota(jnp.int32, (tv.shape[0], 4), 1)
    oh = (tv == cols4).astype(jnp.float32)
    h = jnp.dot(oh, te_ref[...], preferred_element_type=jnp.float32)
    u = jnp.dot(agg, wu_ref[...], preferred_element_type=jnp.float32)
    h2 = u * (1.0 / (1.0 + jnp.exp(-u))) + h
    e_col = jnp.sum(h2 * wo_ref[...], axis=1, keepdims=True)
    bv = batch_ref[0]
    cols8 = lax.broadcasted_iota(jnp.int32, (bv.shape[0], NSYS), 1)
    msk = bv == cols8
    e_sel = jnp.where(msk, e_col, 0.0)
    part = jnp.sum(e_sel, axis=0)[None, :]

    @pl.when(i == 0)
    def _():
        out_ref[...] = jnp.zeros_like(out_ref)

    out_ref[...] += part


def _tc_tail(r2, types3, batch3, type_embed, W_edge, W_msg, W_upd, W_outT):
    bn = 1024
    grid = (NPTC // bn,)
    return pl.pallas_call(
        _tc_tail_kernel,
        grid=grid,
        in_specs=[
            pl.BlockSpec((2, bn, 4 * NB), lambda i: (0, i, 0)),
            pl.BlockSpec((1, bn, 1), lambda i: (i, 0, 0)),
            pl.BlockSpec((1, bn, 1), lambda i: (i, 0, 0)),
            pl.BlockSpec((4, D), lambda i: (0, 0)),
            pl.BlockSpec((NB, D), lambda i: (0, 0)),
            pl.BlockSpec((D, D), lambda i: (0, 0)),
            pl.BlockSpec((D, D), lambda i: (0, 0)),
            pl.BlockSpec((1, D), lambda i: (0, 0)),
        ],
        out_specs=pl.BlockSpec((1, NSYS), lambda i: (0, 0)),
        out_shape=jax.ShapeDtypeStruct((1, NSYS), jnp.float32),
        compiler_params=pltpu.CompilerParams(
            dimension_semantics=("arbitrary",)),
    )(r2, types3, batch3, type_embed, W_edge, W_msg, W_upd, W_outT)


@jax.jit
def kernel(positions, atomic_numbers, edge_index, batch, type_embed, W_edge,
           W_msg, W_upd, W_out):
    z = atomic_numbers
    t = jnp.where(z == 1, 0, jnp.where(z == 6, 1, jnp.where(z == 7, 2, 3)))
    t = t.astype(jnp.int32)

    r2 = _sc_edge_pass(positions, t, edge_index[0], edge_index[1])
    r2 = r2.reshape(2, NPTC, 4 * NB)

    types3 = jnp.zeros((NPTC,), jnp.int32).at[:N].set(t).reshape(
        NPTC // 1024, 1024, 1)
    batch3 = jnp.full((NPTC,), 127, jnp.int32).at[:N].set(batch).reshape(
        NPTC // 1024, 1024, 1)

    energy = _tc_tail(r2, types3, batch3, type_embed, W_edge, W_msg, W_upd,
                      W_out.reshape(1, D))
    return energy[0]


# speedup vs baseline: 1.5257x; 1.5257x over previous
import numpy as np
import jax
import jax.numpy as jnp
from jax import lax
from jax.experimental import pallas as pl
from jax.experimental.pallas import tpu as pltpu
from jax.experimental.pallas import tpu_sc as plsc

N = 10000
E = 320000
D = 128
NB = 8
NSYS = 8
RMAX = 5.0

NPTC = 10240
NROW = 4 * NPTC
TRASH = 4 * N
NW = 32
EPW = E // NW
UNROLL = 5
NITER = EPW // (16 * UNROLL)
RB = 4
ROWS_PER_TILE = NROW // 16


def _sc_edge_kernel(posT_h, types_h, src_h, dst_h, zeros_h, out_h,
                    posx, posy, posz, types, srcb, dstb, rbfb, rowb, racc):
    cid = lax.axis_index("c")
    sid = lax.axis_index("s")
    wid = sid * 2 + cid

    pltpu.sync_copy(zeros_h, racc.at[pl.ds(sid * ROWS_PER_TILE, ROWS_PER_TILE)])
    pltpu.sync_copy(posT_h.at[0], posx)
    pltpu.sync_copy(posT_h.at[1], posy)
    pltpu.sync_copy(posT_h.at[2], posz)
    pltpu.sync_copy(types_h, types)
    base = pl.multiple_of(wid * EPW, 8)
    pltpu.sync_copy(src_h.at[pl.ds(base, EPW)], srcb)
    pltpu.sync_copy(dst_h.at[pl.ds(base, EPW)], dstb)
    plsc.subcore_barrier()

    zero16 = jnp.zeros((16,), jnp.int32)
    pi_over_r = jnp.float32(np.pi / RMAX)
    two_over_pi = jnp.float32(2.0 / np.pi)
    half_pi = jnp.float32(np.pi / 2)
    rmaxf = jnp.float32(RMAX)
    trash16 = jnp.full((16,), TRASH, jnp.int32)
    ringm = jnp.full((16,), RB * 128 - 1, jnp.int32)
    m127 = jnp.full((16,), 127, jnp.int32)

    def body(it, c_in):
        vals = []
        for k in range(UNROLL):
            g0 = it * (16 * UNROLL) + k * 16
            sv = srcb[pl.ds(g0, 16)]
            dv = dstb[pl.ds(g0, 16)]
            xs = plsc.load_gather(posx, [sv])
            ys = plsc.load_gather(posy, [sv])
            zs = plsc.load_gather(posz, [sv])
            xd = plsc.load_gather(posx, [dv])
            yd = plsc.load_gather(posy, [dv])
            zd = plsc.load_gather(posz, [dv])
            tv = plsc.load_gather(types, [sv])

            dx = xd - xs
            dy = yd - ys
            dz = zd - zs
            d2 = dx * dx + dy * dy + dz * dz + jnp.float32(1e-12)
            yi = jnp.int32(0x5F3759DF) - lax.shift_right_arithmetic(
                plsc.bitcast(d2, jnp.int32), 1)
            ry = plsc.bitcast(yi, jnp.float32)
            ry = ry * (1.5 - 0.5 * d2 * ry * ry)
            ry = ry * (1.5 - 0.5 * d2 * ry * ry)
            ry = ry * (1.5 - 0.5 * d2 * ry * ry)
            d = d2 * ry
            keep = d < rmaxf
            theta = jnp.minimum(d, rmaxf) * pi_over_r
            q = (theta * two_over_pi + 0.5).astype(jnp.int32)
            r = theta - q.astype(jnp.float32) * half_pi
            r2 = r * r
            sr = r + r * r2 * (jnp.float32(-1.6666654611e-1) + r2 *
                               (jnp.float32(8.3321608736e-3) + r2 *
                                jnp.float32(-1.9515295891e-4)))
            cr = 1.0 - 0.5 * r2 + r2 * r2 * (
                jnp.float32(4.166664568298827e-2) + r2 *
                (jnp.float32(-1.388731625493765e-3) + r2 *
                 jnp.float32(2.443315711809948e-5)))
            q1 = q == 1
            q0 = q == 0
            s1 = jnp.where(q0, sr, jnp.where(q1, cr, -sr))
            c1 = jnp.where(q0, cr, jnp.where(q1, -sr, -cr))
            g = (0.5 * (c1 + 1.0)) * ry
            twoc = c1 + c1
            rbf = []
            sk_1, sk = s1, twoc * s1
            rbf.append(s1 * g)
            rbf.append(sk * g)
            for _ in range(2, NB):
                sk_1, sk = sk, twoc * sk - sk_1
                rbf.append(sk * g)
            vals.append((keep, rbf, dv * 4 + tv))

        c = c_in
        for k in range(UNROLL):
            keep, rbf, rows_v = vals[k]
            mi = keep.astype(jnp.int32)
            cs = plsc.cumsum(mi)
            pos_v = c + cs - 1
            pr = pos_v & ringm
            for b in range(NB):
                plsc.store_scatter(rbfb, [pr, jnp.full((16,), b, jnp.int32)],
                                   rbf[b], mask=keep)
            plsc.store_scatter(rowb,
                               [lax.shift_right_logical(pr, 7), pr & m127],
                               rows_v, mask=keep)
            c = c + cs[15]

        @pl.when(lax.shift_right_logical(c, 7)
                 > lax.shift_right_logical(c_in, 7))
        def _():
            slot = lax.shift_right_logical(c_in, 7) & (RB - 1)
            pltpu.sync_copy(rbfb.at[pl.ds(slot * 128, 128)],
                            racc.at[rowb.at[slot]], add=True)

        return c

    cnt = lax.fori_loop(0, NITER, body, jnp.int32(0))

    @pl.when((cnt & 127) != 0)
    def _():
        lane = lax.iota(jnp.int32, 16)
        roundup = (cnt + 127) & jnp.int32(~127)
        for i in range(8):
            p = cnt + lane + i * 16
            prp = p & ringm
            plsc.store_scatter(rowb,
                               [lax.shift_right_logical(prp, 7), prp & m127],
                               trash16, mask=p < roundup)
        slot = lax.shift_right_logical(cnt, 7) & (RB - 1)
        pltpu.sync_copy(rbfb.at[pl.ds(slot * 128, 128)],
                        racc.at[rowb.at[slot]], add=True)

    plsc.subcore_barrier()
    pltpu.sync_copy(racc.at[pl.ds(sid * ROWS_PER_TILE, ROWS_PER_TILE)],
                    out_h.at[cid, pl.ds(sid * ROWS_PER_TILE, ROWS_PER_TILE)])


def _sc_edge_pass(posT, types, src, dst):
    zeros = jnp.zeros((ROWS_PER_TILE, NB), jnp.float32)
    mesh = plsc.VectorSubcoreMesh(core_axis_name="c", subcore_axis_name="s")
    k = pl.kernel(
        _sc_edge_kernel,
        out_type=jax.ShapeDtypeStruct((2, NROW, NB), jnp.float32),
        mesh=mesh,
        compiler_params=pltpu.CompilerParams(needs_layout_passes=False,
                                             use_tc_tiling_on_sc=False),
        scratch_types=[
            pltpu.VMEM((N,), jnp.float32),
            pltpu.VMEM((N,), jnp.float32),
            pltpu.VMEM((N,), jnp.float32),
            pltpu.VMEM((N,), jnp.int32),
            pltpu.VMEM((EPW,), jnp.int32),
            pltpu.VMEM((EPW,), jnp.int32),
            pltpu.VMEM((RB * 128, NB), jnp.float32),
            pltpu.VMEM((RB, 128), jnp.int32),
            pltpu.VMEM_SHARED((NROW, NB), jnp.float32),
        ],
    )
    return k(posT, types, src, dst, zeros)


def _tc_tail_kernel(r2_ref, types_ref, batch_ref, te_ref, we_ref, wm_ref,
                    wu_ref, wo_ref, out_ref):
    i = pl.program_id(0)
    rblk = r2_ref[0] + r2_ref[1]
    hm4 = jnp.dot(te_ref[...], wm_ref[...], preferred_element_type=jnp.float32)
    we = we_ref[...]
    wflat = jnp.concatenate([we * hm4[t:t + 1, :] for t in range(4)], axis=0)
    agg = jnp.dot(rblk, wflat, preferred_element_type=jnp.float32)
    tv = types_ref[0]
    cols4 = lax.broadcasted_iota(jnp.int32, (tv.shape[0], 4), 1)
    oh = (tv == cols4).astype(jnp.float32)
    h = jnp.dot(oh, te_ref[...], preferred_element_type=jnp.float32)
    u = jnp.dot(agg, wu_ref[...], preferred_element_type=jnp.float32)
    h2 = u * (1.0 / (1.0 + jnp.exp(-u))) + h
    e_col = jnp.sum(h2 * wo_ref[...], axis=1, keepdims=True)
    bv = batch_ref[0]
    cols8 = lax.broadcasted_iota(jnp.int32, (bv.shape[0], NSYS), 1)
    msk = bv == cols8
    e_sel = jnp.where(msk, e_col, 0.0)
    part = jnp.sum(e_sel, axis=0)[None, :]

    @pl.when(i == 0)
    def _():
        out_ref[...] = jnp.zeros_like(out_ref)

    out_ref[...] += part


def _tc_tail(r2, types3, batch3, type_embed, W_edge, W_msg, W_upd, W_outT):
    bn = 1024
    grid = (NPTC // bn,)
    return pl.pallas_call(
        _tc_tail_kernel,
        grid=grid,
        in_specs=[
            pl.BlockSpec((2, bn, 4 * NB), lambda i: (0, i, 0)),
            pl.BlockSpec((1, bn, 1), lambda i: (i, 0, 0)),
            pl.BlockSpec((1, bn, 1), lambda i: (i, 0, 0)),
            pl.BlockSpec((4, D), lambda i: (0, 0)),
            pl.BlockSpec((NB, D), lambda i: (0, 0)),
            pl.BlockSpec((D, D), lambda i: (0, 0)),
            pl.BlockSpec((D, D), lambda i: (0, 0)),
            pl.BlockSpec((1, D), lambda i: (0, 0)),
        ],
        out_specs=pl.BlockSpec((1, NSYS), lambda i: (0, 0)),
        out_shape=jax.ShapeDtypeStruct((1, NSYS), jnp.float32),
        compiler_params=pltpu.CompilerParams(
            dimension_semantics=("arbitrary",)),
    )(r2, types3, batch3, type_embed, W_edge, W_msg, W_upd, W_outT)


@jax.jit
def kernel(positions, atomic_numbers, edge_index, batch, type_embed, W_edge,
           W_msg, W_upd, W_out):
    z = atomic_numbers
    t = jnp.where(z == 1, 0, jnp.where(z == 6, 1, jnp.where(z == 7, 2, 3)))
    t = t.astype(jnp.int32)

    r2 = _sc_edge_pass(positions.T, t, edge_index[0], edge_index[1])
    r2 = r2.reshape(2, NPTC, 4 * NB)

    types3 = jnp.zeros((NPTC,), jnp.int32).at[:N].set(t).reshape(
        NPTC // 1024, 1024, 1)
    batch3 = jnp.full((NPTC,), 127, jnp.int32).at[:N].set(batch).reshape(
        NPTC // 1024, 1024, 1)

    energy = _tc_tail(r2, types3, batch3, type_embed, W_edge, W_msg, W_upd,
                      W_out.reshape(1, D))
    return energy[0]
